# Initial kernel scaffold; baseline (speedup 1.0000x reference)
import jax
import jax.numpy as jnp
from jax.experimental import pallas as pl


def kernel(objectness, pred_bbox_deltas, anchors):
    raise NotImplementedError("write your pallas kernel here")


# speedup vs baseline: 21.8527x; 21.8527x over previous
import numpy as np
import jax
import jax.numpy as jnp
from jax.experimental import pallas as pl

PRE_NMS_TOP_N = 2000
POST_NMS_TOP_N = 1000
NMS_THRESH = 0.7
SCORE_THRESH = 0.0
MIN_SIZE = 1.0
IMG_H, IMG_W = 800.0, 1333.0
BBOX_XFORM_CLIP = float(np.log(1000.0 / 16.0))
PAD = 2048


def _decode_kernel(lg_ref, dx_ref, dy_ref, dw_ref, dh_ref,
                   ax1_ref, ay1_ref, ax2_ref, ay2_ref,
                   x1_ref, y1_ref, x2_ref, y2_ref, sc_ref):
    ax1 = ax1_ref[...]
    ay1 = ay1_ref[...]
    ax2 = ax2_ref[...]
    ay2 = ay2_ref[...]
    wa = ax2 - ax1
    ha = ay2 - ay1
    cxa = ax1 + 0.5 * wa
    cya = ay1 + 0.5 * ha
    dw = jnp.minimum(dw_ref[...], BBOX_XFORM_CLIP)
    dh = jnp.minimum(dh_ref[...], BBOX_XFORM_CLIP)
    pcx = dx_ref[...] * wa + cxa
    pcy = dy_ref[...] * ha + cya
    pw = jnp.exp(dw) * wa
    ph = jnp.exp(dh) * ha
    x1 = jnp.clip(pcx - 0.5 * pw, 0.0, IMG_W)
    y1 = jnp.clip(pcy - 0.5 * ph, 0.0, IMG_H)
    x2 = jnp.clip(pcx + 0.5 * pw, 0.0, IMG_W)
    y2 = jnp.clip(pcy + 0.5 * ph, 0.0, IMG_H)
    scores = jax.nn.sigmoid(lg_ref[...])
    valid = ((x2 - x1) >= MIN_SIZE) & ((y2 - y1) >= MIN_SIZE) & (scores > SCORE_THRESH)
    x1_ref[...] = x1
    y1_ref[...] = y1
    x2_ref[...] = x2
    y2_ref[...] = y2
    sc_ref[...] = jnp.where(valid, scores, -1.0)


def _nms_kernel(x1_ref, y1_ref, x2_ref, y2_ref, sc_ref, out_ref):
    x1 = x1_ref[...]
    y1 = y1_ref[...]
    x2 = x2_ref[...]
    y2 = y2_ref[...]
    shape = x1.shape
    area = (x2 - x1) * (y2 - y1)
    lane = jax.lax.broadcasted_iota(jnp.int32, shape, 1)

    def body(i, keep):
        sel = lane == i
        neg = jnp.float32(-1e30)
        x1i = jnp.max(jnp.where(sel, x1, neg), axis=1, keepdims=True)
        y1i = jnp.max(jnp.where(sel, y1, neg), axis=1, keepdims=True)
        x2i = jnp.max(jnp.where(sel, x2, neg), axis=1, keepdims=True)
        y2i = jnp.max(jnp.where(sel, y2, neg), axis=1, keepdims=True)
        ki = jnp.max(jnp.where(sel, keep, 0.0), axis=1, keepdims=True)
        areai = (x2i - x1i) * (y2i - y1i)
        w = jnp.clip(jnp.minimum(x2i, x2) - jnp.maximum(x1i, x1), 0.0, None)
        h = jnp.clip(jnp.minimum(y2i, y2) - jnp.maximum(y1i, y1), 0.0, None)
        inter = w * h
        iou = inter / (areai + area - inter + 1e-9)
        sup = (iou > NMS_THRESH) & (lane > i) & (ki > 0.5)
        return jnp.where(sup, 0.0, keep)

    keep = jax.lax.fori_loop(0, PRE_NMS_TOP_N, body, jnp.ones(shape, jnp.float32))
    out_ref[...] = jnp.where(keep > 0.5, sc_ref[...], -1.0)


def kernel(objectness, pred_bbox_deltas, anchors):
    B = objectness.shape[0]
    top_logits, top_idx = jax.lax.top_k(objectness, PRE_NMS_TOP_N)
    d = jnp.take_along_axis(pred_bbox_deltas, top_idx[..., None], axis=1)
    a = anchors[top_idx]

    f32 = jnp.float32
    outs = pl.pallas_call(
        _decode_kernel,
        out_shape=[jax.ShapeDtypeStruct((B, PRE_NMS_TOP_N), f32)] * 5,
    )(top_logits,
      d[..., 0], d[..., 1], d[..., 2], d[..., 3],
      a[..., 0], a[..., 1], a[..., 2], a[..., 3])
    x1, y1, x2, y2, scores = outs

    order = jnp.argsort(-scores, axis=1)
    x1s = jnp.take_along_axis(x1, order, axis=1)
    y1s = jnp.take_along_axis(y1, order, axis=1)
    x2s = jnp.take_along_axis(x2, order, axis=1)
    y2s = jnp.take_along_axis(y2, order, axis=1)
    scs = jnp.take_along_axis(scores, order, axis=1)

    padn = PAD - PRE_NMS_TOP_N
    pad0 = ((0, 0), (0, padn))
    x1p = jnp.pad(x1s, pad0)
    y1p = jnp.pad(y1s, pad0)
    x2p = jnp.pad(x2s, pad0)
    y2p = jnp.pad(y2s, pad0)
    scp = jnp.pad(scs, pad0, constant_values=-1.0)

    final = pl.pallas_call(
        _nms_kernel,
        out_shape=jax.ShapeDtypeStruct((B, PAD), f32),
    )(x1p, y1p, x2p, y2p, scp)

    final = final[:, :PRE_NMS_TOP_N]
    _, keep_idx = jax.lax.top_k(final, POST_NMS_TOP_N)
    boxes = jnp.stack([x1s, y1s, x2s, y2s], axis=-1)
    return jnp.take_along_axis(boxes, keep_idx[..., None], axis=1)
